# baseline (speedup 1.0000x reference)
import functools

import jax
import jax.numpy as jnp
from jax import lax
from jax.experimental import pallas as pl
from jax.experimental.pallas import tpu as pltpu
from jax.experimental.pallas import tpu_sc as plsc


def _sc_gather(table, idx):
    V, D = table.shape
    B = idx.shape[0]
    info = plsc.get_sparse_core_info()
    NC, NS = info.num_cores, info.num_subcores
    NW = NC * NS
    b_per_w = B // NW
    mesh = plsc.VectorSubcoreMesh(core_axis_name="c", subcore_axis_name="s")

    @functools.partial(
        pl.kernel,
        mesh=mesh,
        out_type=jax.ShapeDtypeStruct((B, D), jnp.float32),
        scratch_types=[
            pltpu.VMEM((b_per_w,), jnp.int32),
            pltpu.VMEM((b_per_w, D), jnp.float32),
            pltpu.SemaphoreType.DMA,
        ],
        compiler_params=pltpu.CompilerParams(use_tc_tiling_on_sc=False),
    )
    def gather_k(table_hbm, idx_hbm, out_hbm, idx_v, rows_v, sem):
        wid = lax.axis_index("s") * NC + lax.axis_index("c")
        base = wid * b_per_w
        pltpu.sync_copy(idx_hbm.at[pl.ds(base, b_per_w)], idx_v)
        pltpu.async_copy(table_hbm.at[idx_v], rows_v, sem).wait()
        pltpu.sync_copy(rows_v, out_hbm.at[pl.ds(base, b_per_w)])

    return gather_k(table, idx)


_BNV = 2048


def _matmul_body(wb_ref, lat_ref, out_ref):
    out_ref[...] = lax.dot_general(
        wb_ref[...], lat_ref[...],
        (((0,), (1,)), ((), ())),
        preferred_element_type=jnp.float32,
    )


def _tc_project(latent, W, b):
    B, D = latent.shape
    V = W.shape[0]
    wb = jnp.concatenate([W, b[:, None]], axis=1).T
    lat_ext = jnp.concatenate(
        [latent, jnp.ones((B, 1), jnp.float32)], axis=1)
    grid = pl.cdiv(V, _BNV)
    return pl.pallas_call(
        _matmul_body,
        grid=(grid,),
        in_specs=[
            pl.BlockSpec((D + 1, _BNV), lambda i: (0, i)),
            pl.BlockSpec((B, D + 1), lambda i: (0, 0)),
        ],
        out_specs=pl.BlockSpec((_BNV, B), lambda i: (i, 0)),
        out_shape=jax.ShapeDtypeStruct((V, B), jnp.float32),
        compiler_params=pltpu.CompilerParams(
            vmem_limit_bytes=110 * 1024 * 1024,
        ),
    )(wb, lat_ext)


def kernel(inputs, emb_table, W, b):
    idx = inputs.astype(jnp.int32)
    latent = _sc_gather(emb_table, idx)
    return _tc_project(latent, W, b).T


# speedup vs baseline: 1.0006x; 1.0006x over previous
import functools

import jax
import jax.numpy as jnp
from jax import lax
from jax.experimental import pallas as pl
from jax.experimental.pallas import tpu as pltpu
from jax.experimental.pallas import tpu_sc as plsc


def _sc_gather(table, idx):
    V, D = table.shape
    B = idx.shape[0]
    info = plsc.get_sparse_core_info()
    NC, NS = info.num_cores, info.num_subcores
    NW = NC * NS
    b_per_w = B // NW
    mesh = plsc.VectorSubcoreMesh(core_axis_name="c", subcore_axis_name="s")

    @functools.partial(
        pl.kernel,
        mesh=mesh,
        out_type=jax.ShapeDtypeStruct((B, D), jnp.float32),
        scratch_types=[
            pltpu.VMEM((b_per_w,), jnp.int32),
            pltpu.VMEM((b_per_w, D), jnp.float32),
            pltpu.SemaphoreType.DMA,
        ],
        compiler_params=pltpu.CompilerParams(use_tc_tiling_on_sc=False),
    )
    def gather_k(table_hbm, idx_hbm, out_hbm, idx_v, rows_v, sem):
        wid = lax.axis_index("s") * NC + lax.axis_index("c")
        base = wid * b_per_w
        pltpu.sync_copy(idx_hbm.at[pl.ds(base, b_per_w)], idx_v)
        pltpu.async_copy(table_hbm.at[idx_v], rows_v, sem).wait()
        pltpu.sync_copy(rows_v, out_hbm.at[pl.ds(base, b_per_w)])

    return gather_k(table, idx)


_BNV = 2048


def _matmul_body(wb_ref, lat_ref, out_ref):
    out_ref[...] = lax.dot_general(
        wb_ref[...], lat_ref[...],
        (((0,), (1,)), ((), ())),
        preferred_element_type=jnp.float32,
    )


def _tc_project(latent, W, b):
    B, D = latent.shape
    V = W.shape[0]
    K = 24
    wb = jnp.concatenate(
        [W, b[:, None], jnp.zeros((V, K - D - 1), jnp.float32)], axis=1).T
    lat_ext = jnp.concatenate(
        [latent, jnp.ones((B, 1), jnp.float32),
         jnp.zeros((B, K - D - 1), jnp.float32)], axis=1)
    grid = pl.cdiv(V, _BNV)
    return pl.pallas_call(
        _matmul_body,
        grid=(grid,),
        in_specs=[
            pl.BlockSpec((K, _BNV), lambda i: (0, i)),
            pl.BlockSpec((B, K), lambda i: (0, 0)),
        ],
        out_specs=pl.BlockSpec((_BNV, B), lambda i: (i, 0)),
        out_shape=jax.ShapeDtypeStruct((V, B), jnp.float32),
        compiler_params=pltpu.CompilerParams(
            vmem_limit_bytes=110 * 1024 * 1024,
        ),
    )(wb, lat_ext)


def kernel(inputs, emb_table, W, b):
    idx = inputs.astype(jnp.int32)
    latent = _sc_gather(emb_table, idx)
    return _tc_project(latent, W, b).T


# speedup vs baseline: 1.0406x; 1.0400x over previous
import functools

import jax
import jax.numpy as jnp
from jax import lax
from jax.experimental import pallas as pl
from jax.experimental.pallas import tpu as pltpu
from jax.experimental.pallas import tpu_sc as plsc


def _sc_gather(table, idx):
    V, D = table.shape
    B = idx.shape[0]
    info = plsc.get_sparse_core_info()
    NC, NS = info.num_cores, info.num_subcores
    NW = NC * NS
    b_per_w = B // NW
    mesh = plsc.VectorSubcoreMesh(core_axis_name="c", subcore_axis_name="s")

    @functools.partial(
        pl.kernel,
        mesh=mesh,
        out_type=jax.ShapeDtypeStruct((B, D), jnp.float32),
        scratch_types=[
            pltpu.VMEM((b_per_w,), jnp.int32),
            pltpu.VMEM((b_per_w, D), jnp.float32),
            pltpu.SemaphoreType.DMA,
        ],
        compiler_params=pltpu.CompilerParams(use_tc_tiling_on_sc=False),
    )
    def gather_k(table_hbm, idx_hbm, out_hbm, idx_v, rows_v, sem):
        wid = lax.axis_index("s") * NC + lax.axis_index("c")
        base = wid * b_per_w
        pltpu.sync_copy(idx_hbm.at[pl.ds(base, b_per_w)], idx_v)
        pltpu.async_copy(table_hbm.at[idx_v], rows_v, sem).wait()
        pltpu.sync_copy(rows_v, out_hbm.at[pl.ds(base, b_per_w)])

    return gather_k(table, idx)


_BNV = 2048


def _matmul_body(wb_ref, lat_ref, out_ref):
    out_ref[...] = lax.dot_general(
        wb_ref[...], lat_ref[...],
        (((0,), (1,)), ((), ())),
        preferred_element_type=jnp.float32,
    )


def _tc_project(latent, W, b):
    B, D = latent.shape
    V = W.shape[0]
    K = 24
    wb = jnp.concatenate(
        [W, b[:, None], jnp.zeros((V, K - D - 1), jnp.float32)], axis=1).T
    lat_ext = jnp.concatenate(
        [latent, jnp.ones((B, 1), jnp.float32),
         jnp.zeros((B, K - D - 1), jnp.float32)], axis=1)
    grid = pl.cdiv(V, _BNV)
    return pl.pallas_call(
        _matmul_body,
        grid=(grid,),
        in_specs=[
            pl.BlockSpec((K, _BNV), lambda i: (0, i)),
            pl.BlockSpec((B, K), lambda i: (0, 0)),
        ],
        out_specs=pl.BlockSpec((_BNV, B), lambda i: (i, 0)),
        out_shape=jax.ShapeDtypeStruct((V, B), jnp.float32),
        compiler_params=pltpu.CompilerParams(
            vmem_limit_bytes=110 * 1024 * 1024,
            allow_input_fusion=[True, False],
        ),
    )(wb, lat_ext)


def kernel(inputs, emb_table, W, b):
    idx = inputs.astype(jnp.int32)
    latent = _sc_gather(emb_table, idx)
    return _tc_project(latent, W, b).T
